# baseline (speedup 1.0000x reference)
import jax
import jax.numpy as jnp
from jax.experimental import pallas as pl
from jax.experimental.pallas import tpu as pltpu

_NUM_MODES = 65536
_DIM = 128
_BATCH = 1024
_BK = 2048
_NKB = _NUM_MODES // _BK


def _body(q_ref, k_ref, v_ref, attn_ref, out_ref, acc_ref, l_ref):
    p = pl.program_id(0)
    kb = pl.program_id(1)

    @pl.when((p == 0) & (kb == 0))
    def _init():
        acc_ref[...] = jnp.zeros_like(acc_ref)
        l_ref[...] = jnp.zeros_like(l_ref)

    q = q_ref[...]
    qn = q * jax.lax.rsqrt(jnp.maximum(jnp.sum(q * q, axis=1, keepdims=True), 1e-24))
    k = k_ref[...]
    kn = k * jax.lax.rsqrt(jnp.maximum(jnp.sum(k * k, axis=1, keepdims=True), 1e-24))
    sims = jax.lax.dot_general(
        qn, kn, (((1,), (1,)), ((), ())),
        preferred_element_type=jnp.float32,
        precision=jax.lax.Precision.HIGHEST,
    )
    e = jnp.exp(sims - 1.0)

    @pl.when(p == 0)
    def _pass0():
        l_ref[...] = l_ref[...] + jnp.sum(e, axis=1, keepdims=True)
        acc_ref[...] = acc_ref[...] + jnp.dot(
            e, v_ref[...],
            preferred_element_type=jnp.float32,
            precision=jax.lax.Precision.HIGHEST,
        )

    @pl.when(p == 1)
    def _pass1():
        inv = 1.0 / l_ref[:, 0:1]
        attn_ref[...] = e * inv

        @pl.when(kb == _NKB - 1)
        def _final():
            out_ref[...] = acc_ref[...] * inv


@jax.jit
def kernel(query, keys, values):
    attn, readout = pl.pallas_call(
        _body,
        grid=(2, _NKB),
        in_specs=[
            pl.BlockSpec((_BATCH, _DIM), lambda p, k: (0, 0)),
            pl.BlockSpec((_BK, _DIM), lambda p, k: (k, 0)),
            pl.BlockSpec((_BK, _DIM), lambda p, k: (jax.lax.select(p == 0, k, 0), 0)),
        ],
        out_specs=[
            pl.BlockSpec((_BATCH, _BK), lambda p, k: (0, jax.lax.select(p == 0, 0, k))),
            pl.BlockSpec((_BATCH, _DIM), lambda p, k: (0, 0)),
        ],
        out_shape=[
            jax.ShapeDtypeStruct((_BATCH, _NUM_MODES), jnp.float32),
            jax.ShapeDtypeStruct((_BATCH, _DIM), jnp.float32),
        ],
        scratch_shapes=[
            pltpu.VMEM((_BATCH, _DIM), jnp.float32),
            pltpu.VMEM((_BATCH, 128), jnp.float32),
        ],
        compiler_params=pltpu.CompilerParams(
            dimension_semantics=("arbitrary", "arbitrary"),
        ),
    )(query, keys, values)
    return (readout, attn)


# speedup vs baseline: 3.3073x; 3.3073x over previous
import jax
import jax.numpy as jnp
from jax.experimental import pallas as pl
from jax.experimental.pallas import tpu as pltpu

_NUM_MODES = 65536
_DIM = 128
_BATCH = 1024
_BK = 2048
_NKB = _NUM_MODES // _BK


def _body(q_ref, k_ref, v_ref, attn_ref, out_ref, acc_ref, l_ref):
    p = pl.program_id(0)
    kb = pl.program_id(1)

    @pl.when((p == 0) & (kb == 0))
    def _init():
        acc_ref[...] = jnp.zeros_like(acc_ref)
        l_ref[...] = jnp.zeros_like(l_ref)

    q = q_ref[...]
    qn = (q * jax.lax.rsqrt(jnp.maximum(jnp.sum(q * q, axis=1, keepdims=True), 1e-24))).astype(jnp.bfloat16)
    k = k_ref[...]
    kn = (k * jax.lax.rsqrt(jnp.maximum(jnp.sum(k * k, axis=1, keepdims=True), 1e-24))).astype(jnp.bfloat16)
    sims = jax.lax.dot_general(
        qn, kn, (((1,), (1,)), ((), ())),
        preferred_element_type=jnp.float32,
    )
    e = jnp.exp(sims - 1.0)

    @pl.when(p == 0)
    def _pass0():
        l_ref[...] = l_ref[...] + jnp.sum(e, axis=1, keepdims=True)
        acc_ref[...] = acc_ref[...] + jnp.dot(
            e.astype(jnp.bfloat16), v_ref[...].astype(jnp.bfloat16),
            preferred_element_type=jnp.float32,
        )

    @pl.when(p == 1)
    def _pass1():
        inv = 1.0 / l_ref[:, 0:1]
        attn_ref[...] = e * inv

        @pl.when(kb == _NKB - 1)
        def _final():
            out_ref[...] = acc_ref[...] * inv


@jax.jit
def kernel(query, keys, values):
    attn, readout = pl.pallas_call(
        _body,
        grid=(2, _NKB),
        in_specs=[
            pl.BlockSpec((_BATCH, _DIM), lambda p, k: (0, 0)),
            pl.BlockSpec((_BK, _DIM), lambda p, k: (k, 0)),
            pl.BlockSpec((_BK, _DIM), lambda p, k: (jax.lax.select(p == 0, k, 0), 0)),
        ],
        out_specs=[
            pl.BlockSpec((_BATCH, _BK), lambda p, k: (0, jax.lax.select(p == 0, 0, k))),
            pl.BlockSpec((_BATCH, _DIM), lambda p, k: (0, 0)),
        ],
        out_shape=[
            jax.ShapeDtypeStruct((_BATCH, _NUM_MODES), jnp.float32),
            jax.ShapeDtypeStruct((_BATCH, _DIM), jnp.float32),
        ],
        scratch_shapes=[
            pltpu.VMEM((_BATCH, _DIM), jnp.float32),
            pltpu.VMEM((_BATCH, 128), jnp.float32),
        ],
        compiler_params=pltpu.CompilerParams(
            dimension_semantics=("arbitrary", "arbitrary"),
        ),
    )(query, keys, values)
    return (readout, attn)


# speedup vs baseline: 3.3589x; 1.0156x over previous
import jax
import jax.numpy as jnp
from jax.experimental import pallas as pl
from jax.experimental.pallas import tpu as pltpu

_NUM_MODES = 65536
_DIM = 128
_BATCH = 1024
_BK = 2048
_NKB = _NUM_MODES // _BK


def _body(q_ref, k_ref, v_ref, attn_ref, out_ref, acc_ref, l_ref, kn_ref):
    p = pl.program_id(0)
    kb = pl.program_id(1)

    @pl.when((p == 0) & (kb == 0))
    def _init():
        acc_ref[...] = jnp.zeros_like(acc_ref)
        l_ref[...] = jnp.zeros_like(l_ref)

    q = q_ref[...]
    qn = (q * jax.lax.rsqrt(jnp.maximum(jnp.sum(q * q, axis=1, keepdims=True), 1e-24))).astype(jnp.bfloat16)

    @pl.when(p == 0)
    def _norm_keys():
        k = k_ref[...]
        kn_ref[pl.ds(kb * _BK, _BK), :] = (
            k * jax.lax.rsqrt(jnp.maximum(jnp.sum(k * k, axis=1, keepdims=True), 1e-24))
        ).astype(jnp.bfloat16)

    kn = kn_ref[pl.ds(kb * _BK, _BK), :]
    sims = jax.lax.dot_general(
        qn, kn, (((1,), (1,)), ((), ())),
        preferred_element_type=jnp.float32,
    )
    e = jnp.exp(sims - 1.0)

    @pl.when(p == 0)
    def _pass0():
        l_ref[...] = l_ref[...] + jnp.sum(e, axis=1, keepdims=True)
        acc_ref[...] = acc_ref[...] + jnp.dot(
            e.astype(jnp.bfloat16), v_ref[...].astype(jnp.bfloat16),
            preferred_element_type=jnp.float32,
        )

    @pl.when(p == 1)
    def _pass1():
        inv = 1.0 / l_ref[:, 0:1]
        attn_ref[...] = e * inv

        @pl.when(kb == _NKB - 1)
        def _final():
            out_ref[...] = acc_ref[...] * inv


@jax.jit
def kernel(query, keys, values):
    attn, readout = pl.pallas_call(
        _body,
        grid=(2, _NKB),
        in_specs=[
            pl.BlockSpec((_BATCH, _DIM), lambda p, k: (0, 0)),
            pl.BlockSpec((_BK, _DIM), lambda p, k: (jax.lax.select(p == 0, k, 0), 0)),
            pl.BlockSpec((_BK, _DIM), lambda p, k: (jax.lax.select(p == 0, k, 0), 0)),
        ],
        out_specs=[
            pl.BlockSpec((_BATCH, _BK), lambda p, k: (0, jax.lax.select(p == 0, 0, k))),
            pl.BlockSpec((_BATCH, _DIM), lambda p, k: (0, 0)),
        ],
        out_shape=[
            jax.ShapeDtypeStruct((_BATCH, _NUM_MODES), jnp.float32),
            jax.ShapeDtypeStruct((_BATCH, _DIM), jnp.float32),
        ],
        scratch_shapes=[
            pltpu.VMEM((_BATCH, _DIM), jnp.float32),
            pltpu.VMEM((_BATCH, 128), jnp.float32),
            pltpu.VMEM((_NUM_MODES, _DIM), jnp.bfloat16),
        ],
        compiler_params=pltpu.CompilerParams(
            dimension_semantics=("arbitrary", "arbitrary"),
        ),
    )(query, keys, values)
    return (readout, attn)
